# baseline (speedup 1.0000x reference)
import functools

import jax
import jax.numpy as jnp
from jax import lax
from jax.experimental import pallas as pl
from jax.experimental.pallas import tpu as pltpu
from jax.experimental.pallas import tpu_sc as plsc

B = 4096
F = 26
V = 100000
D = 16
H = 400

NF = F - 2
NC = 2
NS = 16
L = 16
NW = NC * NS
RPW = B // NW


def _sc_gather_body(xt_hbm, tbl_hbm, out_hbm, idx_v, rows_v, gsem, osem):
    wid = lax.axis_index("s") * NC + lax.axis_index("c")
    base = wid * RPW

    pltpu.sync_copy(xt_hbm.at[:, pl.ds(base, RPW)], idx_v)

    for f in range(NF):
        off = (f + 2) * V
        for c in range(RPW // L):
            sl = (f, pl.ds(c * L, L))
            idx_v[sl] = idx_v[sl] + off

    gathers = [
        pltpu.async_copy(tbl_hbm.at[idx_v.at[f]], rows_v.at[f], gsem)
        for f in range(NF)
    ]
    for g in gathers:
        g.wait()

    writes = [
        pltpu.async_copy(
            rows_v.at[f],
            out_hbm.at[pl.ds(base, RPW), pl.ds(f * D, D)],
            osem,
        )
        for f in range(NF)
    ]
    for w in writes:
        w.wait()


@functools.lru_cache(maxsize=None)
def _sc_gather():
    return pl.kernel(
        _sc_gather_body,
        out_type=jax.ShapeDtypeStruct((B, NF * D), jnp.float32),
        mesh=plsc.VectorSubcoreMesh(core_axis_name="c", subcore_axis_name="s",
                                    num_cores=NC, num_subcores=NS),
        compiler_params=pltpu.CompilerParams(use_tc_tiling_on_sc=False),
        scratch_types=[
            pltpu.VMEM((NF, RPW), jnp.int32),
            pltpu.VMEM((NF, RPW, D), jnp.float32),
            pltpu.SemaphoreType.DMA,
            pltpu.SemaphoreType.DMA,
        ],
    )


def _mlp_body(emb_ref, x2_ref, t01_ref, w1_ref, b1_ref, g1_ref, be1_ref,
              w2_ref, b2_ref, g2_ref, be2_ref, out_ref):
    emb = emb_ref[...]
    x2 = x2_ref[...]
    t01 = t01_ref[...]

    e0 = x2[:, 0:1] * t01[0:1, :]
    e1 = x2[:, 1:2] * t01[1:2, :]
    h = (jnp.dot(e0, w1_ref[:D, :], preferred_element_type=jnp.float32)
         + jnp.dot(e1, w1_ref[D:2 * D, :], preferred_element_type=jnp.float32)
         + jnp.dot(emb, w1_ref[2 * D:, :], preferred_element_type=jnp.float32)
         + b1_ref[...])

    m1 = jnp.mean(h, axis=0, keepdims=True)
    c1 = h - m1
    v1 = jnp.mean(c1 * c1, axis=0, keepdims=True)
    h = c1 * lax.rsqrt(v1 + 1e-5) * g1_ref[...] + be1_ref[...]

    h = jnp.dot(h, w2_ref[...], preferred_element_type=jnp.float32) + b2_ref[...]
    m2 = jnp.mean(h, axis=0, keepdims=True)
    c2 = h - m2
    v2 = jnp.mean(c2 * c2, axis=0, keepdims=True)
    out_ref[...] = c2 * lax.rsqrt(v2 + 1e-5) * g2_ref[...] + be2_ref[...]


_mlp = pl.pallas_call(
    _mlp_body,
    out_shape=jax.ShapeDtypeStruct((B, H), jnp.float32),
)


def kernel(x, tables, W1, b1, g1, be1, W2, b2, g2, be2):
    tbl = tables.reshape(F * V, D)
    xt = x[:, 2:].T
    emb = _sc_gather()(xt, tbl)
    x2 = x[:, :2].astype(jnp.float32)
    t01 = tables[:2, 0, :]
    return _mlp(emb, x2, t01, W1, b1, g1, be1, W2, b2, g2, be2)


# speedup vs baseline: 3.9006x; 3.9006x over previous
import functools

import jax
import jax.numpy as jnp
from jax import lax
from jax.experimental import pallas as pl
from jax.experimental.pallas import tpu as pltpu
from jax.experimental.pallas import tpu_sc as plsc

B = 4096
F = 26
V = 100000
D = 16
H = 400

NF = F - 2
NC = 2
NS = 16
L = 16
NW = NC * NS

NFH = NF // 2
TPW = 24
WMAIN = TPW * 128
VMAIN = NS * 2 * WMAIN
NREM = 13
VSCAN = VMAIN + NREM * 128
NTAIL = V - VSCAN

GF = 4
SH = GF * D * B


def _sc_body(xt_hbm, tbl_hbm, out_hbm, xv, blk, blk2, dflat, abuf,
             rowbuf, zbuf, mv, mb, mv2, mb2, sh, sema, semb):
    sid = lax.axis_index("s")
    cid = lax.axis_index("c")
    iota = lax.iota(jnp.int32, L)
    one16 = jnp.ones((L,), jnp.int32)
    zero16 = jnp.zeros((L,), jnp.int32)

    def zinit(j, c):
        zbuf[pl.ds(pl.multiple_of(j * L, L), L)] = jnp.zeros((L,), jnp.float32)
        return c
    lax.fori_loop(0, B // L, zinit, 0)

    lo2 = pl.multiple_of(jnp.where(sid < NREM, VMAIN + sid * 128, 0), 128)

    def chunk_loop(block, mvr, mbr, n, flg, width):
        def cbody(j, c):
            j16 = pl.multiple_of(j * L, L)
            mv16 = mvr[pl.ds(j16, L)]
            mb16 = mbr[pl.ds(j16, L)]
            valid = (iota + j * L) < n
            vloc = jnp.clip(mv16, 0, width - 1)
            for d in range(D):
                row_d = plsc.load_gather(
                    block, [jnp.full((L,), d, jnp.int32), vloc], mask=valid)
                dflat[pl.ds(d * L, L)] = row_d
                addr_d = jnp.where(valid, mb16 + (flg * D + d) * B, SH + iota)
                abuf[pl.ds(d * L, L)] = addr_d
            pltpu.sync_copy(dflat.at[pl.ds(0, 128)],
                            sh.at[abuf.at[pl.ds(0, 128)]])
            pltpu.sync_copy(dflat.at[pl.ds(128, 128)],
                            sh.at[abuf.at[pl.ds(128, 128)]])
            return c
        lax.fori_loop(0, (n + L - 1) // L, cbody, 0)

    for grp in range(NFH // GF):
        share = SH // NS
        for k in range(share // B):
            pltpu.sync_copy(zbuf, sh.at[pl.ds(sid * share + k * B, B)])
        pltpu.sync_copy(zbuf.at[pl.ds(0, L)], sh.at[pl.ds(SH, L)])
        plsc.subcore_barrier()

        def step(it, carry):
            flg = it // 2
            half = it % 2
            fg = cid * NFH + grp * GF + flg
            lo = pl.multiple_of(sid * 2 * WMAIN + half * WMAIN, 128)
            rem = (sid < NREM) & (half == 0)
            pltpu.sync_copy(xt_hbm.at[fg, :], xv)
            r0 = pl.multiple_of((fg + 2) * D, 8)
            cpa = pltpu.async_copy(
                tbl_hbm.at[pl.ds(r0, D), pl.ds(lo, WMAIN)], blk, sema)
            cpb = pltpu.async_copy(
                tbl_hbm.at[pl.ds(r0, D), pl.ds(lo2, 128)], blk2, semb)
            hi2 = jnp.where(rem, lo2 + 128, 0)

            def sbody(c, carry2):
                off, off2 = carry2
                c16 = pl.multiple_of(c * L, L)
                v16 = xv[pl.ds(c16, L)]
                b16 = iota + c * L
                m = (v16 >= lo) & (v16 < lo + WMAIN)
                pos = plsc.cumsum(jnp.where(m, one16, zero16)) - 1 + off
                plsc.store_scatter(mv, [pos], v16 - lo, mask=m)
                plsc.store_scatter(mb, [pos], b16, mask=m)
                m2 = (v16 >= lo2) & (v16 < hi2)
                pos2 = plsc.cumsum(jnp.where(m2, one16, zero16)) - 1 + off2
                plsc.store_scatter(mv2, [pos2], v16 - lo2, mask=m2)
                plsc.store_scatter(mb2, [pos2], b16, mask=m2)
                return (off + plsc.all_reduce_population_count(m),
                        off2 + plsc.all_reduce_population_count(m2))

            off, off2 = lax.fori_loop(0, B // L, sbody, (zero16, zero16))
            n = jnp.max(off)
            n2 = jnp.max(off2)
            cpa.wait()
            cpb.wait()
            chunk_loop(blk, mv, mb, n, flg, WMAIN)
            chunk_loop(blk2, mv2, mb2, n2, flg, 128)
            return carry

        lax.fori_loop(0, 2 * GF, step, 0)
        plsc.subcore_barrier()

        s8 = sid % 8
        for k in range(8):
            r = s8 * 8 + k
            pltpu.sync_copy(sh.at[pl.ds(r * B, B)], rowbuf.at[k, :])
        row0 = pl.multiple_of(cid * NFH * D + grp * GF * D + s8 * 8, 8)
        pltpu.sync_copy(rowbuf, out_hbm.at[pl.ds(row0, 8), :])
        plsc.subcore_barrier()


@functools.lru_cache(maxsize=None)
def _sc_gather():
    return pl.kernel(
        _sc_body,
        out_type=jax.ShapeDtypeStruct((NF * D, B), jnp.float32),
        mesh=plsc.VectorSubcoreMesh(core_axis_name="c", subcore_axis_name="s",
                                    num_cores=NC, num_subcores=NS),
        compiler_params=pltpu.CompilerParams(needs_layout_passes=False),
        scratch_types=[
            pltpu.VMEM((B,), jnp.int32),
            pltpu.VMEM((D, WMAIN), jnp.float32),
            pltpu.VMEM((D, 128), jnp.float32),
            pltpu.VMEM((D * L,), jnp.float32),
            pltpu.VMEM((D * L,), jnp.int32),
            pltpu.VMEM((8, B), jnp.float32),
            pltpu.VMEM((B,), jnp.float32),
            pltpu.VMEM((B,), jnp.int32),
            pltpu.VMEM((B,), jnp.int32),
            pltpu.VMEM((B,), jnp.int32),
            pltpu.VMEM((B,), jnp.int32),
            pltpu.VMEM_SHARED((SH + L,), jnp.float32),
            pltpu.SemaphoreType.DMA,
            pltpu.SemaphoreType.DMA,
        ],
    )


def _mlp_body(embt_ref, x_ref, t01_ref, tail_ref, w1_ref, b1_ref,
              g1_ref, be1_ref, w2_ref, b2_ref, g2_ref, be2_ref, out_ref):
    x2 = x_ref[:, 0:2].astype(jnp.float32)
    t01 = t01_ref[...]

    h = lax.dot_general(embt_ref[...], w1_ref[2 * D:, :],
                        (((0,), (0,)), ((), ())),
                        preferred_element_type=jnp.float32)
    e0 = x2[:, 0:1] * t01[0:1, :]
    e1 = x2[:, 1:2] * t01[1:2, :]
    h = (h
         + jnp.dot(e0, w1_ref[:D, :], preferred_element_type=jnp.float32)
         + jnp.dot(e1, w1_ref[D:2 * D, :], preferred_element_type=jnp.float32)
         + b1_ref[...])

    iota32 = lax.broadcasted_iota(jnp.int32, (1, NTAIL), 1)
    for f in range(NF):
        mf = jnp.dot(tail_ref[pl.ds(f * NTAIL, NTAIL), :],
                     w1_ref[pl.ds(2 * D + f * D, D), :],
                     preferred_element_type=jnp.float32)
        oh = (x_ref[:, 2 + f:3 + f] == VSCAN + iota32).astype(jnp.float32)
        h = h + jnp.dot(oh, mf, preferred_element_type=jnp.float32)

    m1 = jnp.mean(h, axis=0, keepdims=True)
    c1 = h - m1
    v1 = jnp.mean(c1 * c1, axis=0, keepdims=True)
    h = c1 * lax.rsqrt(v1 + 1e-5) * g1_ref[...] + be1_ref[...]

    h = jnp.dot(h, w2_ref[...], preferred_element_type=jnp.float32) + b2_ref[...]
    m2 = jnp.mean(h, axis=0, keepdims=True)
    c2 = h - m2
    v2 = jnp.mean(c2 * c2, axis=0, keepdims=True)
    out_ref[...] = c2 * lax.rsqrt(v2 + 1e-5) * g2_ref[...] + be2_ref[...]


_mlp = pl.pallas_call(
    _mlp_body,
    out_shape=jax.ShapeDtypeStruct((B, H), jnp.float32),
)


def kernel(x, tables, W1, b1, g1, be1, W2, b2, g2, be2):
    t416 = jnp.transpose(tables, (0, 2, 1)).reshape(F * D, V)
    xt = x[:, 2:].T
    embt = _sc_gather()(xt, t416)
    t01 = tables[:2, 0, :]
    tail = tables[2:, VSCAN:, :].reshape(NF * NTAIL, D)
    return _mlp(embt, x, t01, tail, W1, b1, g1, be1, W2, b2, g2, be2)
